# Initial kernel scaffold; baseline (speedup 1.0000x reference)
import jax
import jax.numpy as jnp
from jax.experimental import pallas as pl


def kernel(right, left, iter, left_weights):
    raise NotImplementedError("write your pallas kernel here")


# speedup vs baseline: 1.9775x; 1.9775x over previous
import functools

import jax
import jax.numpy as jnp
from jax import lax
from jax.experimental import pallas as pl
from jax.experimental.pallas import tpu as pltpu
from jax.experimental.pallas import tpu_sc as plsc

BATCH = 4096
NUM_STAGES = 10
CODE_LEN = 1024
ROW = (NUM_STAGES + 1) * CODE_LEN
CLIP = 20.0
LANES = 16
NCHUNK = CODE_LEN // LANES


def _minsum(x, y):
    return jnp.sign(x) * jnp.sign(y) * jnp.minimum(jnp.abs(x), jnp.abs(y))


def _clip(x):
    return jnp.minimum(jnp.maximum(x, -CLIP), CLIP)


def _make_sc_kernel():
    info = plsc.get_sparse_core_info()
    nc, ns = info.num_cores, info.num_subcores
    nw = nc * ns
    rows_per_w = BATCH // nw

    mesh = plsc.VectorSubcoreMesh(core_axis_name="c", subcore_axis_name="s")

    @functools.partial(
        pl.kernel,
        out_type=jax.ShapeDtypeStruct((BATCH * ROW,), jnp.float32),
        mesh=mesh,
        scratch_types=[
            pltpu.VMEM((CODE_LEN,), jnp.float32),
            pltpu.VMEM((NUM_STAGES * CODE_LEN,), jnp.float32),
            pltpu.VMEM(((NUM_STAGES + 1) * CODE_LEN,), jnp.float32),
            pltpu.VMEM((2 * NUM_STAGES * LANES,), jnp.float32),
        ],
    )
    def sc_kernel(right_hbm, left_hbm, w_hbm, out_hbm, cur, rbuf, obuf, wbuf):
        wid = lax.axis_index("s") * nc + lax.axis_index("c")
        pltpu.sync_copy(w_hbm, wbuf)
        lanes = lax.iota(jnp.int32, 16)

        def row_body(k, carry):
            roff = (wid * rows_per_w + k) * ROW
            pltpu.sync_copy(left_hbm.at[pl.ds(roff + NUM_STAGES * CODE_LEN, CODE_LEN)], cur)
            pltpu.sync_copy(right_hbm.at[pl.ds(roff, NUM_STAGES * CODE_LEN)], rbuf)

            def clip_body(c, carry):
                b = c * LANES
                obuf[pl.ds(NUM_STAGES * CODE_LEN + b, LANES)] = _clip(cur[pl.ds(b, LANES)])
                return carry

            lax.fori_loop(0, NCHUNK, clip_body, 0)

            for i in reversed(range(NUM_STAGES)):
                w0 = wbuf[pl.ds(i * LANES, LANES)]
                w1 = wbuf[pl.ds((NUM_STAGES + i) * LANES, LANES)]
                if i >= 4:
                    sh = i - 4
                    db = 1 << sh

                    def pair_body(t, carry, i=i, sh=sh, db=db, w0=w0, w1=w1):
                        mchunk = ((t >> sh) << (sh + 1)) | (t & (db - 1))
                        mb = mchunk * LANES
                        pb = mb + db * LANES
                        cm = cur[pl.ds(mb, LANES)]
                        cp = cur[pl.ds(pb, LANES)]
                        rm = rbuf[pl.ds(i * CODE_LEN + mb, LANES)]
                        rp = rbuf[pl.ds(i * CODE_LEN + pb, LANES)]
                        nm = w0 * _minsum(cm, cp + rp)
                        npv = w1 * _minsum(cm, rm) + cp
                        cur[pl.ds(mb, LANES)] = nm
                        cur[pl.ds(pb, LANES)] = npv
                        obuf[pl.ds(i * CODE_LEN + mb, LANES)] = _clip(nm)
                        obuf[pl.ds(i * CODE_LEN + pb, LANES)] = _clip(npv)
                        return carry

                    lax.fori_loop(0, NCHUNK // 2, pair_body, 0)
                else:
                    d = 1 << i
                    xidx = lanes ^ d
                    upper = (lanes & d) == 0

                    def chunk_body(c, carry, i=i, xidx=xidx, upper=upper,
                                   w0=w0, w1=w1):
                        b = c * LANES
                        cc = cur[pl.ds(b, LANES)]
                        rc = rbuf[pl.ds(i * CODE_LEN + b, LANES)]
                        cs = cc.at[xidx].get(mode="promise_in_bounds")
                        rs = rc.at[xidx].get(mode="promise_in_bounds")
                        up = w0 * _minsum(cc, cs + rs)
                        lo = w1 * _minsum(cs, rs) + cc
                        nv = jnp.where(upper, up, lo)
                        cur[pl.ds(b, LANES)] = nv
                        obuf[pl.ds(i * CODE_LEN + b, LANES)] = _clip(nv)
                        return carry

                    lax.fori_loop(0, NCHUNK, chunk_body, 0)

            pltpu.sync_copy(obuf, out_hbm.at[pl.ds(roff, ROW)])
            return carry

        lax.fori_loop(0, rows_per_w, row_body, 0)

    return sc_kernel


_SC_KERNEL = None


def kernel(right, left, iter, left_weights):
    global _SC_KERNEL
    if _SC_KERNEL is None:
        _SC_KERNEL = _make_sc_kernel()
    w = left_weights[iter]
    wv = jnp.broadcast_to(
        w.T[:, :, None], (2, NUM_STAGES, LANES)
    ).astype(jnp.float32).reshape(2 * NUM_STAGES * LANES)
    out = _SC_KERNEL(right.reshape(-1), left.reshape(-1), wv)
    return out.reshape(BATCH, NUM_STAGES + 1, CODE_LEN)


# speedup vs baseline: 2.8850x; 1.4589x over previous
import functools

import jax
import jax.numpy as jnp
from jax import lax
from jax.experimental import pallas as pl
from jax.experimental.pallas import tpu as pltpu
from jax.experimental.pallas import tpu_sc as plsc

BATCH = 4096
NUM_STAGES = 10
CODE_LEN = 1024
RSZ = NUM_STAGES * CODE_LEN
ROW = (NUM_STAGES + 1) * CODE_LEN
CLIP = 20.0
LANES = 16
NCHUNK = CODE_LEN // LANES
SIGN = jnp.int32(-2**31)
MAG = jnp.int32(2**31 - 1)


def _f2i(x):
    return lax.bitcast_convert_type(x, jnp.int32)


def _i2f(x):
    return lax.bitcast_convert_type(x, jnp.float32)


def _minsum(x, y):
    xb = _f2i(x)
    yb = _f2i(y)
    s = (xb ^ yb) & SIGN
    m = _f2i(jnp.minimum(_i2f(xb & MAG), _i2f(yb & MAG)))
    return _i2f(m | s)


def _clip(x):
    return jnp.minimum(jnp.maximum(x, -CLIP), CLIP)


def _make_sc_kernel():
    info = plsc.get_sparse_core_info()
    nc, ns = info.num_cores, info.num_subcores
    nw = nc * ns
    rows_per_w = BATCH // nw

    mesh = plsc.VectorSubcoreMesh(core_axis_name="c", subcore_axis_name="s")

    @functools.partial(
        pl.kernel,
        out_type=jax.ShapeDtypeStruct((BATCH * ROW,), jnp.float32),
        mesh=mesh,
        scratch_types=[
            pltpu.VMEM((2, CODE_LEN), jnp.float32),
            pltpu.VMEM((2, RSZ), jnp.float32),
            pltpu.VMEM((2, ROW), jnp.float32),
            pltpu.VMEM((2 * NUM_STAGES * LANES,), jnp.float32),
            pltpu.SemaphoreType.DMA,
            pltpu.SemaphoreType.DMA,
            pltpu.SemaphoreType.DMA,
            pltpu.SemaphoreType.DMA,
            pltpu.SemaphoreType.DMA,
            pltpu.SemaphoreType.DMA,
        ],
    )
    def sc_kernel(right_hbm, left_hbm, w_hbm, out_hbm,
                  cur, rbuf, obuf, wbuf,
                  inr0, inr1, inl0, inl1, out0, out1):
        wid = lax.axis_index("s") * nc + lax.axis_index("c")
        base = wid * rows_per_w
        inr = (inr0, inr1)
        inl = (inl0, inl1)
        osem = (out0, out1)
        pltpu.sync_copy(w_hbm, wbuf)
        lanes = lax.iota(jnp.int32, 16)
        w0s = [wbuf[pl.ds(i * LANES, LANES)] for i in range(NUM_STAGES)]
        w1s = [wbuf[pl.ds((NUM_STAGES + i) * LANES, LANES)]
               for i in range(NUM_STAGES)]

        def issue_in(s, k):
            roff = (base + k) * ROW
            pltpu.async_copy(right_hbm.at[pl.ds(roff, RSZ)], rbuf.at[s], inr[s])
            pltpu.async_copy(left_hbm.at[pl.ds(roff + RSZ, CODE_LEN)],
                             cur.at[s], inl[s])

        issue_in(0, 0)
        issue_in(1, 1)

        def do_row(s, k):
            roff = (base + k) * ROW
            pltpu.make_async_copy(right_hbm.at[pl.ds(roff, RSZ)],
                                  rbuf.at[s], inr[s]).wait()
            pltpu.make_async_copy(left_hbm.at[pl.ds(roff + RSZ, CODE_LEN)],
                                  cur.at[s], inl[s]).wait()

            @pl.when(k >= 2)
            def _():
                pltpu.make_async_copy(obuf.at[s], out_hbm.at[pl.ds(roff, ROW)],
                                      osem[s]).wait()

            @plsc.parallel_loop(0, NCHUNK, unroll=4)
            def clip_body(c):
                b = c * LANES
                obuf[s, pl.ds(RSZ + b, LANES)] = _clip(cur[s, pl.ds(b, LANES)])

            for i in reversed(range(NUM_STAGES)):
                w0 = w0s[i]
                w1 = w1s[i]
                if i >= 4:
                    sh = i - 4
                    db = 1 << sh

                    @plsc.parallel_loop(0, NCHUNK // 2, unroll=2)
                    def pair_body(t, i=i, sh=sh, db=db, w0=w0, w1=w1):
                        mchunk = ((t >> sh) << (sh + 1)) | (t & (db - 1))
                        mb = mchunk * LANES
                        pb = mb + db * LANES
                        cm = cur[s, pl.ds(mb, LANES)]
                        cp = cur[s, pl.ds(pb, LANES)]
                        rm = rbuf[s, pl.ds(i * CODE_LEN + mb, LANES)]
                        rp = rbuf[s, pl.ds(i * CODE_LEN + pb, LANES)]
                        nm = w0 * _minsum(cm, cp + rp)
                        npv = w1 * _minsum(cm, rm) + cp
                        cur[s, pl.ds(mb, LANES)] = nm
                        cur[s, pl.ds(pb, LANES)] = npv
                        obuf[s, pl.ds(i * CODE_LEN + mb, LANES)] = _clip(nm)
                        obuf[s, pl.ds(i * CODE_LEN + pb, LANES)] = _clip(npv)
                else:
                    d = 1 << i
                    xidx = lanes ^ d
                    upper = (lanes & d) == 0

                    @plsc.parallel_loop(0, NCHUNK, unroll=2)
                    def chunk_body(c, i=i, xidx=xidx, upper=upper,
                                   w0=w0, w1=w1):
                        b = c * LANES
                        cc = cur[s, pl.ds(b, LANES)]
                        rc = rbuf[s, pl.ds(i * CODE_LEN + b, LANES)]
                        cs = cc.at[xidx].get(mode="promise_in_bounds")
                        rs = rc.at[xidx].get(mode="promise_in_bounds")
                        up = w0 * _minsum(cc, cs + rs)
                        lo = w1 * _minsum(cs, rs) + cc
                        nv = jnp.where(upper, up, lo)
                        cur[s, pl.ds(b, LANES)] = nv
                        obuf[s, pl.ds(i * CODE_LEN + b, LANES)] = _clip(nv)

            pltpu.async_copy(obuf.at[s], out_hbm.at[pl.ds(roff, ROW)], osem[s])

            @pl.when(k + 2 < rows_per_w)
            def _():
                issue_in(s, k + 2)

        def row_pair(j, carry):
            do_row(0, 2 * j)
            do_row(1, 2 * j + 1)
            return carry

        lax.fori_loop(0, rows_per_w // 2, row_pair, 0)

        for s, k in ((0, rows_per_w - 2), (1, rows_per_w - 1)):
            roff = (base + k) * ROW
            pltpu.make_async_copy(obuf.at[s], out_hbm.at[pl.ds(roff, ROW)],
                                  osem[s]).wait()

    return sc_kernel


_SC_KERNEL = None


def kernel(right, left, iter, left_weights):
    global _SC_KERNEL
    if _SC_KERNEL is None:
        _SC_KERNEL = _make_sc_kernel()
    w = left_weights[iter]
    wv = jnp.broadcast_to(
        w.T[:, :, None], (2, NUM_STAGES, LANES)
    ).astype(jnp.float32).reshape(2 * NUM_STAGES * LANES)
    out = _SC_KERNEL(right.reshape(-1), left.reshape(-1), wv)
    return out.reshape(BATCH, NUM_STAGES + 1, CODE_LEN)
